# baseline (speedup 1.0000x reference)
import functools

import jax
import jax.numpy as jnp
from jax import lax
from jax.experimental import pallas as pl
from jax.experimental.pallas import tpu as pltpu
from jax.experimental.pallas import tpu_sc as plsc

D_MODEL = 64
SEQ_LEN = 200
BATCH = 1024
VOC = 1000000
NTOK = BATCH * SEQ_LEN

_info = plsc.get_sparse_core_info()
NC, NS = _info.num_cores, _info.num_subcores
NW = NC * NS
VLANES = 16

TCB = 512
TCGRID = (VOC + TCB - 1) // TCB
VOCP = TCGRID * TCB


def _tc_compact_body(tabt_ref, out_ref):
    x = tabt_ref[...]
    y = jnp.transpose(x)
    out_ref[:, 0:D_MODEL] = y[0:TCB // 2, :]
    out_ref[:, D_MODEL:2 * D_MODEL] = y[TCB // 2:TCB, :]


_compact_table = pl.pallas_call(
    _tc_compact_body,
    grid=(TCGRID,),
    in_specs=[pl.BlockSpec((D_MODEL, TCB), lambda i: (0, i))],
    out_specs=pl.BlockSpec((TCB // 2, 2 * D_MODEL), lambda i: (i, 0)),
    out_shape=jax.ShapeDtypeStruct((VOCP // 2, 2 * D_MODEL), jnp.float32),
)


CHUNK = SEQ_LEN
NCHUNK = NTOK // NW // CHUNK
SUB = 40
NSUB = CHUNK // SUB
NCOL = D_MODEL // VLANES


@functools.partial(
    pl.kernel,
    out_type=jax.ShapeDtypeStruct((BATCH, SEQ_LEN, D_MODEL), jnp.float32),
    mesh=plsc.VectorSubcoreMesh(core_axis_name="c", subcore_axis_name="s"),
    scratch_types=[
        pltpu.VMEM((SEQ_LEN, D_MODEL), jnp.float32),
        pltpu.VMEM((2, NSUB, SUB), jnp.int32),
        pltpu.VMEM((2, CHUNK, D_MODEL), jnp.float32),
        pltpu.SemaphoreType.DMA,
        pltpu.SemaphoreType.DMA,
    ],
    compiler_params=pltpu.CompilerParams(use_tc_tiling_on_sc=False),
)
def _emb_lookup(idx_hbm, table_hbm, pos_hbm, out_hbm, pos_v, idx_v, rows_v,
                sem_a, sem_b):
    sems = (sem_a, sem_b)
    wid = lax.axis_index("s") * NC + lax.axis_index("c")

    pltpu.sync_copy(pos_hbm, pos_v)

    def fire(k, b):
        base = wid * (NCHUNK * CHUNK) + k * CHUNK
        for j in range(NSUB):
            pltpu.sync_copy(idx_hbm.at[pl.ds(base + j * SUB, SUB)],
                            idx_v.at[b].at[j])
        for j in range(NSUB):
            pltpu.async_copy(
                table_hbm.at[idx_v.at[b].at[j]],
                rows_v.at[b].at[pl.ds(j * SUB, SUB)],
                sems[b],
            )

    def drain(b):
        pltpu.make_async_copy(
            table_hbm.at[pl.ds(0, CHUNK)], rows_v.at[b], sems[b]
        ).wait()

    def add_pos(b):
        rb = rows_v.at[b]

        def body(r, carry):
            for c in range(NCOL):
                sl = pl.ds(c * VLANES, VLANES)
                rb[r, sl] += pos_v[r, sl]
            return carry

        lax.fori_loop(0, CHUNK, body, 0)

    def store(k, b):
        seq = wid * NCHUNK + k
        pltpu.sync_copy(rows_v.at[b], out_hbm.at[seq])

    fire(0, 0)
    for k in range(NCHUNK):
        b = k & 1
        if k + 1 < NCHUNK:
            fire(k + 1, 1 - b)
        drain(b)
        add_pos(b)
        store(k, b)


def kernel(inputs, input_emb_table, positional_emb_table):
    tabt = jnp.transpose(input_emb_table)
    packed = _compact_table(tabt)
    table_lin = packed.reshape(VOCP, D_MODEL)
    v = inputs.astype(jnp.int32).reshape(NTOK)
    idx = (v & ~511) | ((v & 255) << 1) | ((v >> 8) & 1)
    return _emb_lookup(idx, table_lin, positional_emb_table)


# speedup vs baseline: 1.7056x; 1.7056x over previous
import functools

import jax
import jax.numpy as jnp
from jax import lax
from jax.experimental import pallas as pl
from jax.experimental.pallas import tpu as pltpu
from jax.experimental.pallas import tpu_sc as plsc

D_MODEL = 64
SEQ_LEN = 200
BATCH = 1024
VOC = 1000000
NTOK = BATCH * SEQ_LEN

_info = plsc.get_sparse_core_info()
NC, NS = _info.num_cores, _info.num_subcores
NW = NC * NS
VLANES = 16

CHUNK = SEQ_LEN
NCHUNK = NTOK // NW // CHUNK
SUB = 40
NSUB = CHUNK // SUB
NCOL = D_MODEL // VLANES


@functools.partial(
    pl.kernel,
    out_type=jax.ShapeDtypeStruct((BATCH, SEQ_LEN, D_MODEL), jnp.float32),
    mesh=plsc.VectorSubcoreMesh(core_axis_name="c", subcore_axis_name="s"),
    scratch_types=[
        pltpu.VMEM((SEQ_LEN, D_MODEL), jnp.float32),
        pltpu.VMEM((2, NSUB, SUB), jnp.int32),
        pltpu.VMEM((2, CHUNK, D_MODEL), jnp.float32),
        pltpu.SemaphoreType.DMA,
        pltpu.SemaphoreType.DMA,
    ],
    compiler_params=pltpu.CompilerParams(use_tc_tiling_on_sc=False),
)
def _emb_lookup(idx_hbm, table_hbm, pos_hbm, out_hbm, pos_v, idx_v, rows_v,
                sem_a, sem_b):
    sems = (sem_a, sem_b)
    wid = lax.axis_index("s") * NC + lax.axis_index("c")

    pltpu.sync_copy(pos_hbm, pos_v)

    def fire(k, b):
        base = wid * (NCHUNK * CHUNK) + k * CHUNK
        for j in range(NSUB):
            pltpu.sync_copy(idx_hbm.at[pl.ds(base + j * SUB, SUB)],
                            idx_v.at[b].at[j])
        for j in range(NSUB):
            pltpu.async_copy(
                table_hbm.at[idx_v.at[b].at[j]],
                rows_v.at[b].at[pl.ds(j * SUB, SUB)],
                sems[b],
            )

    def drain(b):
        pltpu.make_async_copy(
            table_hbm.at[pl.ds(0, CHUNK)], rows_v.at[b], sems[b]
        ).wait()

    def add_pos(b):
        rb = rows_v.at[b]

        def body(r, carry):
            for c in range(NCOL):
                sl = pl.ds(c * VLANES, VLANES)
                rb[r, sl] += pos_v[r, sl]
            return carry

        lax.fori_loop(0, CHUNK, body, 0)

    def store(k, b):
        seq = wid * NCHUNK + k
        pltpu.sync_copy(rows_v.at[b], out_hbm.at[seq])

    fire(0, 0)
    for k in range(NCHUNK):
        b = k & 1
        if k + 1 < NCHUNK:
            fire(k + 1, 1 - b)
        drain(b)
        add_pos(b)
        store(k, b)


def kernel(inputs, input_emb_table, positional_emb_table):
    idx = inputs.astype(jnp.int32).reshape(NTOK)
    return _emb_lookup(idx, input_emb_table, positional_emb_table)


# speedup vs baseline: 1.8249x; 1.0700x over previous
import functools

import jax
import jax.numpy as jnp
from jax import lax
from jax.experimental import pallas as pl
from jax.experimental.pallas import tpu as pltpu
from jax.experimental.pallas import tpu_sc as plsc

D_MODEL = 64
SEQ_LEN = 200
BATCH = 1024
NTOK = BATCH * SEQ_LEN

_info = plsc.get_sparse_core_info()
NC, NS = _info.num_cores, _info.num_subcores
NW = NC * NS
ROWS_PER_W = NTOK // NW
CHUNK = 320
NCHUNK = ROWS_PER_W // CHUNK
SUB = 40
NSUB = CHUNK // SUB
PEXT = 480
VLANES = 16
NCOL = D_MODEL // VLANES


@functools.partial(
    pl.kernel,
    out_type=jax.ShapeDtypeStruct((NTOK, D_MODEL), jnp.float32),
    mesh=plsc.VectorSubcoreMesh(core_axis_name="c", subcore_axis_name="s"),
    scratch_types=[
        pltpu.VMEM((PEXT, D_MODEL), jnp.float32),
        pltpu.VMEM((2, NSUB, SUB), jnp.int32),
        pltpu.VMEM((2, CHUNK, D_MODEL), jnp.float32),
        pltpu.SemaphoreType.DMA,
        pltpu.SemaphoreType.DMA,
    ],
    compiler_params=pltpu.CompilerParams(use_tc_tiling_on_sc=False),
)
def _emb_lookup(idx_hbm, table_hbm, pos_hbm, out_hbm, pos_v, idx_v, rows_v,
                sem_a, sem_b):
    sems = (sem_a, sem_b)
    wid = lax.axis_index("s") * NC + lax.axis_index("c")

    pltpu.sync_copy(pos_hbm, pos_v.at[pl.ds(0, SEQ_LEN)])
    pltpu.sync_copy(pos_hbm, pos_v.at[pl.ds(SEQ_LEN, SEQ_LEN)])
    pltpu.sync_copy(pos_hbm.at[pl.ds(0, PEXT - 2 * SEQ_LEN)],
                    pos_v.at[pl.ds(2 * SEQ_LEN, PEXT - 2 * SEQ_LEN)])

    def fire(k, b):
        r0 = wid * (ROWS_PER_W // SUB) + k * NSUB
        pltpu.sync_copy(idx_hbm.at[pl.ds(r0, NSUB)], idx_v.at[b])
        for j in range(NSUB):
            pltpu.async_copy(
                table_hbm.at[idx_v.at[b].at[j]],
                rows_v.at[b].at[pl.ds(j * SUB, SUB)],
                sems[b],
            )

    def drain(b):
        pltpu.make_async_copy(
            table_hbm.at[pl.ds(0, CHUNK)], rows_v.at[b], sems[b]
        ).wait()

    def add_pos(k, b):
        rb = rows_v.at[b]
        p0 = (k * CHUNK) % SEQ_LEN

        def body(r, carry):
            for c in range(NCOL):
                sl = pl.ds(c * VLANES, VLANES)
                rb[r, sl] += pos_v[p0 + r, sl]
            return carry

        lax.fori_loop(0, CHUNK, body, 0)

    def store(k, b):
        base = wid * ROWS_PER_W + k * CHUNK
        pltpu.sync_copy(rows_v.at[b], out_hbm.at[pl.ds(base, CHUNK)])

    fire(0, 0)
    for k in range(NCHUNK):
        b = k & 1
        if k + 1 < NCHUNK:
            fire(k + 1, 1 - b)
        drain(b)
        add_pos(k, b)
        store(k, b)


def kernel(inputs, input_emb_table, positional_emb_table):
    idx = inputs.astype(jnp.int32).reshape(NTOK // SUB, SUB)
    out = _emb_lookup(idx, input_emb_table, positional_emb_table)
    return out.reshape(BATCH, SEQ_LEN, D_MODEL)
